# baseline (speedup 1.0000x reference)
import functools

import jax
import jax.numpy as jnp
from jax import lax
from jax.experimental import pallas as pl
from jax.experimental.pallas import tpu as pltpu
from jax.experimental.pallas import tpu_sc as plsc

B = 16384
K = 64
GAMMA = 1.0
NC, NS = 2, 16
NW = NC * NS
CH = B // NW
HCH = CH // 2
GCH = 128
NG = HCH // GCH
L = 16
TS = L + 1


def _rsqrt16(x):
    xc = jnp.maximum(x, jnp.float32(1e-30))
    i = plsc.bitcast(xc, jnp.int32)
    i = jnp.int32(0x5F3759DF) - (i >> 1)
    y = plsc.bitcast(i, jnp.float32)
    half = jnp.float32(0.5) * xc
    for _ in range(3):
        y = y * (jnp.float32(1.5) - half * y * y)
    return y


TCOLS = 8192


def _tc_transpose(tab_t):
    M = tab_t.shape[1]
    nblk = pl.cdiv(M, TCOLS)

    def body(in_ref, eye_ref, out_ref):
        x = in_ref[...]
        hi = x.astype(jnp.bfloat16)
        lo = (x - hi.astype(jnp.float32)).astype(jnp.bfloat16)
        xs = jnp.concatenate([hi, lo], axis=0)
        out_ref[...] = jax.lax.dot_general(
            xs, eye_ref[...], (((0,), (0,)), ((), ())),
            preferred_element_type=jnp.float32)

    out = pl.pallas_call(
        body,
        grid=(nblk,),
        in_specs=[pl.BlockSpec((K, TCOLS), lambda j: (0, j)),
                  pl.BlockSpec((2 * K, K), lambda j: (0, 0))],
        out_specs=pl.BlockSpec((TCOLS, K), lambda j: (j, 0)),
        out_shape=jax.ShapeDtypeStruct((M, K), jnp.float32),
    )(tab_t, jnp.concatenate([jnp.eye(K), jnp.eye(K)]).astype(jnp.bfloat16))
    return out.reshape(M // 2, 2 * K)


def _make_kernel():
    mesh = plsc.VectorSubcoreMesh(core_axis_name="c", subcore_axis_name="s")
    f32 = jnp.float32
    i32 = jnp.int32

    @functools.partial(
        pl.kernel,
        mesh=mesh,
        compiler_params=pltpu.CompilerParams(needs_layout_passes=False),
        out_type=[
            jax.ShapeDtypeStruct((B,), f32),
            jax.ShapeDtypeStruct((B,), f32),
            jax.ShapeDtypeStruct((B,), f32),
        ],
        scratch_types=[
            pltpu.VMEM((NG, GCH), i32),
            pltpu.VMEM((NG, GCH), i32),
            pltpu.VMEM((NG, GCH), i32),
            pltpu.VMEM((NG, GCH), i32),
            pltpu.VMEM((NG, GCH), i32),
            pltpu.VMEM((NG, GCH), i32),
            pltpu.VMEM((HCH, 2 * K), f32),
            pltpu.VMEM((HCH, 2 * K), f32),
            pltpu.VMEM((HCH, 2 * K), f32),
            pltpu.VMEM((L * TS,), f32),
            pltpu.VMEM((CH,), f32),
            pltpu.VMEM((CH,), f32),
            pltpu.VMEM((CH,), f32),
            pltpu.SemaphoreType.DMA,
            pltpu.SemaphoreType.DMA,
            pltpu.SemaphoreType.DMA,
        ],
    )
    def trans_e(h_hbm, l_hbm, t_hbm, ent_hbm, rel_hbm,
                loss_hbm, dtr_hbm, dco_hbm,
                pidx_h, pidx_l, pidx_t, idx_h, idx_l, idx_t,
                rows_h, rows_l, rows_t,
                tbuf, dist_tr, dist_co, loss_v, sem_h, sem_l, sem_t):
        wid = lax.axis_index("s") * NC + lax.axis_index("c")
        iota = lax.iota(i32, L)

        def one_half(batch, half, dist_out):
            irow = batch * (B // GCH) + wid * (CH // GCH) + half * NG
            pltpu.sync_copy(h_hbm.at[pl.ds(irow, NG)], idx_h)
            pltpu.sync_copy(l_hbm.at[pl.ds(irow, NG)], idx_l)
            pltpu.sync_copy(t_hbm.at[pl.ds(irow, NG)], idx_t)
            for j in range(NG):
                for c in range(GCH // L):
                    sl = pl.ds(c * L, L)
                    pidx_h[j, sl] = idx_h[j, sl] >> 1
                    pidx_l[j, sl] = idx_l[j, sl] >> 1
                    pidx_t[j, sl] = idx_t[j, sl] >> 1
            copies = []
            for j in range(NG):
                sl = pl.ds(j * GCH, GCH)
                copies.append(pltpu.async_copy(
                    ent_hbm.at[pidx_h.at[j]], rows_h.at[sl], sem_h))
                copies.append(pltpu.async_copy(
                    rel_hbm.at[pidx_l.at[j]], rows_l.at[sl], sem_l))
                copies.append(pltpu.async_copy(
                    ent_hbm.at[pidx_t.at[j]], rows_t.at[sl], sem_t))
            for c in copies:
                c.wait()

            def group(g, _):
                base = g * L
                jrow = g // (GCH // L)
                jsl = pl.ds((g % (GCH // L)) * L, L)
                hoff = (idx_h[jrow, jsl] & 1) << 6
                loff = (idx_l[jrow, jsl] & 1) << 6
                toff = (idx_t[jrow, jsl] & 1) << 6
                for j in range(L):
                    row = base + j
                    ho = hoff[j]
                    lo = loff[j]
                    to = toff[j]
                    acc = None
                    for c in range(K // L):
                        d = (rows_h[row, pl.ds(ho + c * L, L)]
                             + rows_l[row, pl.ds(lo + c * L, L)]
                             - rows_t[row, pl.ds(to + c * L, L)])
                        sq = d * d
                        acc = sq if acc is None else acc + sq
                    tbuf[pl.ds(j * TS, L)] = acc
                s = None
                for c in range(L):
                    col = plsc.load_gather(tbuf, [iota * TS + c])
                    s = col if s is None else s + col
                dist_out[pl.ds(half * HCH + base, L)] = s * _rsqrt16(s)
                return 0

            lax.fori_loop(0, HCH // L, group, 0)

        for batch, dist in ((0, dist_tr), (1, dist_co)):
            for half in (0, 1):
                one_half(batch, half, dist)

        def loss_group(g, _):
            sl = pl.ds(g * L, L)
            loss_v[sl] = jnp.maximum(
                jnp.float32(0.0), dist_tr[sl] - dist_co[sl] + jnp.float32(GAMMA))
            return 0

        lax.fori_loop(0, CH // L, loss_group, 0)

        out = pl.ds(wid * CH, CH)
        pltpu.sync_copy(loss_v, loss_hbm.at[out])
        pltpu.sync_copy(dist_tr, dtr_hbm.at[out])
        pltpu.sync_copy(dist_co, dco_hbm.at[out])

    return trans_e


_TRANS_E = _make_kernel()


@jax.jit
def kernel(training_triplets, corrupted_triplets, entities_embedding,
           relations_embedding):
    cols = []
    for c in range(3):
        col = jnp.concatenate(
            [training_triplets[:, c], corrupted_triplets[:, c]])
        cols.append(col.reshape(2 * B // GCH, GCH))
    h_idx, l_idx, t_idx = cols
    ent2 = _tc_transpose(jnp.transpose(entities_embedding))
    rel2 = _tc_transpose(jnp.transpose(relations_embedding))
    loss, dist_tr, dist_co = _TRANS_E(h_idx, l_idx, t_idx, ent2, rel2)
    return (loss, dist_tr, dist_co)


# speedup vs baseline: 1.1898x; 1.1898x over previous
import functools

import jax
import jax.numpy as jnp
from jax import lax
from jax.experimental import pallas as pl
from jax.experimental.pallas import tpu as pltpu
from jax.experimental.pallas import tpu_sc as plsc

B = 16384
K = 64
GAMMA = 1.0
NC, NS = 2, 16
NW = NC * NS
CH = B // NW
GCH = 128
NG = CH // GCH
L = 16


def _rsqrt16(x):
    xc = jnp.maximum(x, jnp.float32(1e-30))
    i = plsc.bitcast(xc, jnp.int32)
    i = jnp.int32(0x5F3759DF) - (i >> 1)
    y = plsc.bitcast(i, jnp.float32)
    half = jnp.float32(0.5) * xc
    for _ in range(3):
        y = y * (jnp.float32(1.5) - half * y * y)
    return y


def _make_kernel():
    mesh = plsc.VectorSubcoreMesh(core_axis_name="c", subcore_axis_name="s")
    f32 = jnp.float32

    @functools.partial(
        pl.kernel,
        mesh=mesh,
        compiler_params=pltpu.CompilerParams(
            needs_layout_passes=False, use_tc_tiling_on_sc=False),
        out_type=[
            jax.ShapeDtypeStruct((B,), f32),
            jax.ShapeDtypeStruct((B,), f32),
            jax.ShapeDtypeStruct((B,), f32),
        ],
        scratch_types=[
            pltpu.VMEM((NG, GCH), jnp.int32),
            pltpu.VMEM((NG, GCH), jnp.int32),
            pltpu.VMEM((NG, GCH), jnp.int32),
            pltpu.VMEM((CH, K), f32),
            pltpu.VMEM((CH, K), f32),
            pltpu.VMEM((CH, K), f32),
            pltpu.VMEM((L * L,), f32),
            pltpu.VMEM((CH,), f32),
            pltpu.VMEM((CH,), f32),
            pltpu.VMEM((CH,), f32),
            pltpu.SemaphoreType.DMA,
            pltpu.SemaphoreType.DMA,
            pltpu.SemaphoreType.DMA,
        ],
    )
    def trans_e(h_hbm, l_hbm, t_hbm, ent_hbm, rel_hbm,
                loss_hbm, dtr_hbm, dco_hbm,
                idx_h, idx_l, idx_t, rows_h, rows_l, rows_t,
                tbuf, dist_tr, dist_co, loss_v, sem_h, sem_l, sem_t):
        wid = lax.axis_index("s") * NC + lax.axis_index("c")
        iota = lax.iota(jnp.int32, L)

        def one_batch(batch, dist_out):
            irow = batch * (B // GCH) + wid * NG
            pltpu.sync_copy(h_hbm.at[pl.ds(irow, NG)], idx_h)
            pltpu.sync_copy(l_hbm.at[pl.ds(irow, NG)], idx_l)
            pltpu.sync_copy(t_hbm.at[pl.ds(irow, NG)], idx_t)
            copies = []
            for j in range(NG):
                sl = pl.ds(j * GCH, GCH)
                copies.append(pltpu.async_copy(
                    ent_hbm.at[idx_h.at[j]], rows_h.at[sl], sem_h))
                copies.append(pltpu.async_copy(
                    rel_hbm.at[idx_l.at[j]], rows_l.at[sl], sem_l))
                copies.append(pltpu.async_copy(
                    ent_hbm.at[idx_t.at[j]], rows_t.at[sl], sem_t))
            for c in copies:
                c.wait()

            def group(g, _):
                base = g * L
                for j in range(L):
                    row = base + j
                    acc = None
                    for c in range(K // L):
                        cs = pl.ds(c * L, L)
                        d = rows_h[row, cs] + rows_l[row, cs] - rows_t[row, cs]
                        sq = d * d
                        acc = sq if acc is None else acc + sq
                    tbuf[pl.ds(j * L, L)] = acc
                s = None
                for c in range(L):
                    col = plsc.load_gather(tbuf, [iota * L + c])
                    s = col if s is None else s + col
                dist_out[pl.ds(base, L)] = s * _rsqrt16(s)
                return 0

            lax.fori_loop(0, CH // L, group, 0)

        one_batch(0, dist_tr)
        one_batch(1, dist_co)

        def loss_group(g, _):
            sl = pl.ds(g * L, L)
            loss_v[sl] = jnp.maximum(
                jnp.float32(0.0), dist_tr[sl] - dist_co[sl] + jnp.float32(GAMMA))
            return 0

        lax.fori_loop(0, CH // L, loss_group, 0)

        out = pl.ds(wid * CH, CH)
        pltpu.sync_copy(loss_v, loss_hbm.at[out])
        pltpu.sync_copy(dist_tr, dtr_hbm.at[out])
        pltpu.sync_copy(dist_co, dco_hbm.at[out])

    return trans_e


_TRANS_E = _make_kernel()


@jax.jit
def kernel(training_triplets, corrupted_triplets, entities_embedding,
           relations_embedding):
    cols = []
    for c in range(3):
        col = jnp.concatenate(
            [training_triplets[:, c], corrupted_triplets[:, c]])
        cols.append(col.reshape(2 * B // GCH, GCH))
    h_idx, l_idx, t_idx = cols
    loss, dist_tr, dist_co = _TRANS_E(
        h_idx, l_idx, t_idx, entities_embedding, relations_embedding)
    return (loss, dist_tr, dist_co)
